# baseline (speedup 1.0000x reference)
import functools

import jax
import jax.numpy as jnp
from jax import lax
from jax.experimental import pallas as pl
from jax.experimental.pallas import tpu as pltpu
from jax.experimental.pallas import tpu_sc as plsc

N_NODES = 10000
N_EDGES = 320000
D_FEAT = 128
OUT_FEATS = 128
FAT = 144

NC = 2
NS = 16
NW = NC * NS
EDGES_PER_W = N_EDGES // NW
BLK = 50
NBLK = EDGES_PER_W // BLK
ROWS_PER_TILE = N_NODES // NS
NBUF = 2
LOOKAHEAD = 1


def _sc_segment_sum(fat_feature, src_r, dst_r, zeros_hbm):
    mesh = plsc.VectorSubcoreMesh(core_axis_name="c", subcore_axis_name="s")

    @functools.partial(
        pl.kernel,
        mesh=mesh,
        compiler_params=pltpu.CompilerParams(use_tc_tiling_on_sc=False),
        out_type=jax.ShapeDtypeStruct((NC, N_NODES, FAT), jnp.float32),
        scratch_types=[
            pltpu.VMEM((NBLK, BLK), jnp.int32),
            pltpu.VMEM((NBLK, BLK), jnp.int32),
            pltpu.VMEM_SHARED((N_NODES, FAT), jnp.float32),
            [pltpu.VMEM((BLK, FAT), jnp.float32)] * NBUF,
            [pltpu.SemaphoreType.DMA] * NBUF,
            [pltpu.SemaphoreType.DMA] * NBUF,
        ],
    )
    def kern(fat_hbm, src_hbm, dst_hbm, zero_hbm, out_hbm,
             src_v, dst_v, acc_sh, rows, sem_g, sem_s):
        c = lax.axis_index("c")
        s = lax.axis_index("s")
        wid = s * NC + c

        pltpu.sync_copy(src_hbm.at[wid], src_v)
        pltpu.sync_copy(dst_hbm.at[wid], dst_v)

        for b in range(LOOKAHEAD):
            pltpu.async_copy(fat_hbm.at[src_v.at[b]], rows[b], sem_g[b])

        pltpu.sync_copy(zero_hbm, acc_sh.at[pl.ds(s * ROWS_PER_TILE, ROWS_PER_TILE)])
        plsc.subcore_barrier()

        def outer(k, carry):
            j0 = k * NBUF
            for b in range(NBUF):
                j = j0 + b
                bg = (b + LOOKAHEAD) % NBUF
                pltpu.make_async_copy(fat_hbm.at[src_v.at[j]], rows[b],
                                      sem_g[b]).wait()
                pltpu.async_copy(rows[b], acc_sh.at[dst_v.at[j]], sem_s[b],
                                 add=True)

                @pl.when(j + LOOKAHEAD < NBLK)
                def _():
                    @pl.when(j >= NBUF - LOOKAHEAD)
                    def _():
                        pltpu.make_async_copy(
                            rows[bg], acc_sh.at[dst_v.at[j]], sem_s[bg]).wait()
                    pltpu.async_copy(fat_hbm.at[src_v.at[j + LOOKAHEAD]],
                                     rows[bg], sem_g[bg])
            return carry

        lax.fori_loop(0, NBLK // NBUF, outer, 0)

        for b in range(NBUF):
            pltpu.make_async_copy(rows[b], acc_sh.at[dst_v.at[b]],
                                  sem_s[b]).wait()
        plsc.subcore_barrier()

        pltpu.sync_copy(
            acc_sh.at[pl.ds(s * ROWS_PER_TILE, ROWS_PER_TILE)],
            out_hbm.at[c].at[pl.ds(s * ROWS_PER_TILE, ROWS_PER_TILE)],
        )

    return kern(fat_feature, src_r, dst_r, zeros_hbm)


def _tc_mean_linear(partials, feature, w1t, w2t, b2d):
    blk_rows = 1000
    grid = (N_NODES // blk_rows,)

    def body(p_ref, f_ref, w1_ref, w2_ref, b_ref, o_ref):
        p0 = p_ref[0]
        p1 = p_ref[1]
        summed = p0[:, :D_FEAT] + p1[:, :D_FEAT]
        cnt = p0[:, D_FEAT:D_FEAT + 1] + p1[:, D_FEAT:D_FEAT + 1]
        h = summed / jnp.maximum(cnt, 1.0)
        o_ref[...] = (
            jnp.dot(h, w1_ref[...], preferred_element_type=jnp.float32)
            + jnp.dot(f_ref[...], w2_ref[...], preferred_element_type=jnp.float32)
            + b_ref[...]
        )

    return pl.pallas_call(
        body,
        grid=grid,
        in_specs=[
            pl.BlockSpec((NC, blk_rows, FAT), lambda i: (0, i, 0)),
            pl.BlockSpec((blk_rows, D_FEAT), lambda i: (i, 0)),
            pl.BlockSpec((D_FEAT, OUT_FEATS), lambda i: (0, 0)),
            pl.BlockSpec((D_FEAT, OUT_FEATS), lambda i: (0, 0)),
            pl.BlockSpec((1, OUT_FEATS), lambda i: (0, 0)),
        ],
        out_specs=pl.BlockSpec((blk_rows, OUT_FEATS), lambda i: (i, 0)),
        out_shape=jax.ShapeDtypeStruct((N_NODES, OUT_FEATS), jnp.float32),
    )(partials, feature, w1t, w2t, b2d)


def kernel(feature, edge_index, W, b):
    ei = edge_index.astype(jnp.int32)
    src_r = ei[0].reshape(NW, NBLK, BLK)
    dst_r = ei[1].reshape(NW, NBLK, BLK)

    fat = jnp.zeros((N_NODES, FAT), jnp.float32)
    fat = fat.at[:, :D_FEAT].set(feature)
    fat = fat.at[:, D_FEAT].set(1.0)

    zeros_hbm = jnp.zeros((ROWS_PER_TILE, FAT), jnp.float32)

    partials = _sc_segment_sum(fat, src_r, dst_r, zeros_hbm)

    w1t = W[:, :D_FEAT].T
    w2t = W[:, D_FEAT:].T
    b2d = b.reshape(1, OUT_FEATS)
    return _tc_mean_linear(partials, feature, w1t, w2t, b2d)


# speedup vs baseline: 1.4181x; 1.4181x over previous
import functools

import jax
import jax.numpy as jnp
from jax import lax
from jax.experimental import pallas as pl
from jax.experimental.pallas import tpu as pltpu
from jax.experimental.pallas import tpu_sc as plsc

N_NODES = 10000
N_EDGES = 320000
D_FEAT = 128
OUT_FEATS = 128

NC = 2
NS = 16
NW = NC * NS
EDGES_PER_W = N_EDGES // NW
BLK = 125
NBLK = EDGES_PER_W // BLK
ROWS_PER_TILE = N_NODES // NS
LANE = 16
CNT_STEPS = BLK // LANE
CNT_TAIL = BLK - CNT_STEPS * LANE


def _sc_segment_sum(feature, src_r, dst_r, zero_rows, zero_cnt):
    mesh = plsc.VectorSubcoreMesh(core_axis_name="c", subcore_axis_name="s")

    @functools.partial(
        pl.kernel,
        mesh=mesh,
        compiler_params=pltpu.CompilerParams(
            use_tc_tiling_on_sc=False, needs_layout_passes=False),
        out_type=(
            jax.ShapeDtypeStruct((NC, N_NODES, D_FEAT), jnp.float32),
            jax.ShapeDtypeStruct((NW, N_NODES), jnp.float32),
        ),
        scratch_types=[
            pltpu.VMEM((NBLK, BLK), jnp.int32),
            pltpu.VMEM((NBLK, BLK), jnp.int32),
            pltpu.VMEM((BLK, D_FEAT), jnp.float32),
            pltpu.VMEM((N_NODES,), jnp.float32),
            pltpu.VMEM_SHARED((N_NODES, D_FEAT), jnp.float32),
            pltpu.SemaphoreType.DMA,
        ],
    )
    def kern(feat_hbm, src_hbm, dst_hbm, zrows_hbm, zcnt_hbm,
             out_hbm, cnt_hbm, src_v, dst_v, rows_v, cnt_v, acc_sh, sem):
        c = lax.axis_index("c")
        s = lax.axis_index("s")
        wid = s * NC + c

        pltpu.sync_copy(src_hbm.at[wid], src_v)
        pltpu.sync_copy(dst_hbm.at[wid], dst_v)
        pltpu.sync_copy(zcnt_hbm, cnt_v)
        pltpu.sync_copy(zrows_hbm, acc_sh.at[pl.ds(s * ROWS_PER_TILE, ROWS_PER_TILE)])
        plsc.subcore_barrier()

        ones = jnp.ones((LANE,), jnp.float32)
        tail_mask = lax.iota(jnp.int32, LANE) >= (LANE - CNT_TAIL)

        def blk(j, carry):
            gather = pltpu.async_copy(feat_hbm.at[src_v.at[j]], rows_v, sem)
            for k in range(CNT_STEPS):
                idx = dst_v[j, pl.ds(k * LANE, LANE)]
                plsc.addupdate_scatter(cnt_v, [idx], ones)
            idx = dst_v[j, pl.ds(BLK - LANE, LANE)]
            plsc.addupdate_scatter(cnt_v, [idx], ones, mask=tail_mask)
            gather.wait()
            pltpu.sync_copy(rows_v, acc_sh.at[dst_v.at[j]], add=True)
            return carry

        lax.fori_loop(0, NBLK, blk, 0)

        pltpu.sync_copy(cnt_v, cnt_hbm.at[wid])
        plsc.subcore_barrier()
        pltpu.sync_copy(
            acc_sh.at[pl.ds(s * ROWS_PER_TILE, ROWS_PER_TILE)],
            out_hbm.at[c].at[pl.ds(s * ROWS_PER_TILE, ROWS_PER_TILE)],
        )

    return kern(feature, src_r, dst_r, zero_rows, zero_cnt)


def _tc_mean_linear(partials, counts2d, feature, w1t, w2t, b2d):
    blk_rows = 1000
    grid = (N_NODES // blk_rows,)

    def body(p_ref, c_ref, f_ref, w1_ref, w2_ref, b_ref, o_ref):
        summed = p_ref[0] + p_ref[1]
        cnt = c_ref[...]
        h = summed / jnp.maximum(cnt, 1.0)
        o_ref[...] = (
            jnp.dot(h, w1_ref[...], preferred_element_type=jnp.float32)
            + jnp.dot(f_ref[...], w2_ref[...], preferred_element_type=jnp.float32)
            + b_ref[...]
        )

    return pl.pallas_call(
        body,
        grid=grid,
        in_specs=[
            pl.BlockSpec((NC, blk_rows, D_FEAT), lambda i: (0, i, 0)),
            pl.BlockSpec((blk_rows, 1), lambda i: (i, 0)),
            pl.BlockSpec((blk_rows, D_FEAT), lambda i: (i, 0)),
            pl.BlockSpec((D_FEAT, OUT_FEATS), lambda i: (0, 0)),
            pl.BlockSpec((D_FEAT, OUT_FEATS), lambda i: (0, 0)),
            pl.BlockSpec((1, OUT_FEATS), lambda i: (0, 0)),
        ],
        out_specs=pl.BlockSpec((blk_rows, OUT_FEATS), lambda i: (i, 0)),
        out_shape=jax.ShapeDtypeStruct((N_NODES, OUT_FEATS), jnp.float32),
    )(partials, counts2d, feature, w1t, w2t, b2d)


def kernel(feature, edge_index, W, b):
    ei = edge_index.astype(jnp.int32)
    src_r = ei[0].reshape(NW, NBLK, BLK)
    dst_r = ei[1].reshape(NW, NBLK, BLK)

    zero_rows = jnp.zeros((ROWS_PER_TILE, D_FEAT), jnp.float32)
    zero_cnt = jnp.zeros((N_NODES,), jnp.float32)

    partials, counts = _sc_segment_sum(feature, src_r, dst_r, zero_rows, zero_cnt)
    counts2d = counts.sum(axis=0).reshape(N_NODES, 1)

    w1t = W[:, :D_FEAT].T
    w2t = W[:, D_FEAT:].T
    b2d = b.reshape(1, OUT_FEATS)
    return _tc_mean_linear(partials, counts2d, feature, w1t, w2t, b2d)


# speedup vs baseline: 1.6768x; 1.1825x over previous
import functools

import jax
import jax.numpy as jnp
from jax import lax
from jax.experimental import pallas as pl
from jax.experimental.pallas import tpu as pltpu
from jax.experimental.pallas import tpu_sc as plsc

N_NODES = 10000
N_EDGES = 320000
D_FEAT = 128
OUT_FEATS = 128

NC = 2
NS = 16
NW = NC * NS
EDGES_PER_W = N_EDGES // NW
BLK = 125
NBLK = EDGES_PER_W // BLK
ROWS_PER_TILE = N_NODES // NS
NBUF = 2
NIDX = 4
LANE = 16
CNT_STEPS = BLK // LANE
CNT_TAIL = BLK - CNT_STEPS * LANE


def _sc_segment_sum(feature, idx_r, zero_rows, zero_cnt):
    mesh = plsc.VectorSubcoreMesh(core_axis_name="c", subcore_axis_name="s")

    @functools.partial(
        pl.kernel,
        mesh=mesh,
        compiler_params=pltpu.CompilerParams(
            use_tc_tiling_on_sc=False, needs_layout_passes=False),
        out_type=(
            jax.ShapeDtypeStruct((NC, N_NODES, D_FEAT), jnp.float32),
            jax.ShapeDtypeStruct((NW, N_NODES), jnp.float32),
        ),
        scratch_types=[
            [pltpu.VMEM((2, BLK), jnp.int32)] * NIDX,
            [pltpu.VMEM((BLK, D_FEAT), jnp.float32)] * NBUF,
            pltpu.VMEM((N_NODES,), jnp.float32),
            pltpu.VMEM_SHARED((N_NODES, D_FEAT), jnp.float32),
            [pltpu.SemaphoreType.DMA] * NIDX,
            [pltpu.SemaphoreType.DMA] * NBUF,
            [pltpu.SemaphoreType.DMA] * NBUF,
        ],
    )
    def kern(feat_hbm, idx_hbm, zrows_hbm, zcnt_hbm,
             out_hbm, cnt_hbm, islot, rows, cnt_v, acc_sh,
             sem_i, sem_g, sem_s):
        c = lax.axis_index("c")
        s = lax.axis_index("s")
        wid = s * NC + c
        me = idx_hbm.at[wid]

        for q in range(NIDX - 1):
            pltpu.async_copy(me.at[q], islot[q], sem_i[q])
        pltpu.sync_copy(zcnt_hbm, cnt_v)
        pltpu.make_async_copy(me.at[0], islot[0], sem_i[0]).wait()
        pltpu.async_copy(feat_hbm.at[islot[0].at[0]], rows[0], sem_g[0])
        pltpu.sync_copy(zrows_hbm, acc_sh.at[pl.ds(s * ROWS_PER_TILE, ROWS_PER_TILE)])
        plsc.subcore_barrier()

        ones = jnp.ones((LANE,), jnp.float32)
        tail_mask = lax.iota(jnp.int32, LANE) >= (LANE - CNT_TAIL)

        def outer(k, carry):
            j0 = k * NIDX
            for q in range(NIDX):
                j = j0 + q
                b = q % NBUF
                pltpu.make_async_copy(feat_hbm.at[islot[q].at[0]], rows[b],
                                      sem_g[b]).wait()
                pltpu.async_copy(rows[b], acc_sh.at[islot[q].at[1]], sem_s[b],
                                 add=True)
                for t in range(CNT_STEPS):
                    idx = islot[q][1, pl.ds(t * LANE, LANE)]
                    plsc.addupdate_scatter(cnt_v, [idx], ones)
                idx = islot[q][1, pl.ds(BLK - LANE, LANE)]
                plsc.addupdate_scatter(cnt_v, [idx], ones, mask=tail_mask)

                bn = (q + 1) % NBUF
                qn = (q + 1) % NIDX

                @pl.when(j + 1 < NBLK)
                def _():
                    @pl.when(j >= 1)
                    def _():
                        pltpu.make_async_copy(
                            rows[bn], acc_sh.at[islot[qn].at[1]],
                            sem_s[bn]).wait()
                    pltpu.make_async_copy(me.at[j + 1], islot[qn],
                                          sem_i[qn]).wait()
                    pltpu.async_copy(feat_hbm.at[islot[qn].at[0]], rows[bn],
                                     sem_g[bn])

                @pl.when(j + NIDX - 1 < NBLK)
                def _():
                    qf = (q + NIDX - 1) % NIDX
                    pltpu.async_copy(me.at[j + NIDX - 1], islot[qf],
                                     sem_i[qf])
            return carry

        lax.fori_loop(0, NBLK // NIDX, outer, 0)

        for b in range(NBUF):
            pltpu.make_async_copy(rows[b], acc_sh.at[islot[b].at[1]],
                                  sem_s[b]).wait()
        pltpu.sync_copy(cnt_v, cnt_hbm.at[wid])
        plsc.subcore_barrier()
        pltpu.sync_copy(
            acc_sh.at[pl.ds(s * ROWS_PER_TILE, ROWS_PER_TILE)],
            out_hbm.at[c].at[pl.ds(s * ROWS_PER_TILE, ROWS_PER_TILE)],
        )

    return kern(feature, idx_r, zero_rows, zero_cnt)


def _tc_mean_linear(partials, counts2d, feature, w1t, w2t, b2d):
    blk_rows = 1000
    grid = (N_NODES // blk_rows,)

    def body(p_ref, c_ref, f_ref, w1_ref, w2_ref, b_ref, o_ref):
        summed = p_ref[0] + p_ref[1]
        cnt = c_ref[...]
        h = summed / jnp.maximum(cnt, 1.0)
        o_ref[...] = (
            jnp.dot(h, w1_ref[...], preferred_element_type=jnp.float32)
            + jnp.dot(f_ref[...], w2_ref[...], preferred_element_type=jnp.float32)
            + b_ref[...]
        )

    return pl.pallas_call(
        body,
        grid=grid,
        in_specs=[
            pl.BlockSpec((NC, blk_rows, D_FEAT), lambda i: (0, i, 0)),
            pl.BlockSpec((blk_rows, 1), lambda i: (i, 0)),
            pl.BlockSpec((blk_rows, D_FEAT), lambda i: (i, 0)),
            pl.BlockSpec((D_FEAT, OUT_FEATS), lambda i: (0, 0)),
            pl.BlockSpec((D_FEAT, OUT_FEATS), lambda i: (0, 0)),
            pl.BlockSpec((1, OUT_FEATS), lambda i: (0, 0)),
        ],
        out_specs=pl.BlockSpec((blk_rows, OUT_FEATS), lambda i: (i, 0)),
        out_shape=jax.ShapeDtypeStruct((N_NODES, OUT_FEATS), jnp.float32),
    )(partials, counts2d, feature, w1t, w2t, b2d)


def kernel(feature, edge_index, W, b):
    ei = edge_index.astype(jnp.int32)
    src_r = ei[0].reshape(NW, NBLK, BLK)
    dst_r = ei[1].reshape(NW, NBLK, BLK)
    idx_r = jnp.stack([src_r, dst_r], axis=2)

    zero_rows = jnp.zeros((ROWS_PER_TILE, D_FEAT), jnp.float32)
    zero_cnt = jnp.zeros((N_NODES,), jnp.float32)

    partials, counts = _sc_segment_sum(feature, idx_r, zero_rows, zero_cnt)
    counts2d = counts.sum(axis=0).reshape(N_NODES, 1)

    w1t = W[:, :D_FEAT].T
    w2t = W[:, D_FEAT:].T
    b2d = b.reshape(1, OUT_FEATS)
    return _tc_mean_linear(partials, counts2d, feature, w1t, w2t, b2d)


# speedup vs baseline: 1.7189x; 1.0251x over previous
import functools

import jax
import jax.numpy as jnp
from jax import lax
from jax.experimental import pallas as pl
from jax.experimental.pallas import tpu as pltpu
from jax.experimental.pallas import tpu_sc as plsc

N_NODES = 10000
N_EDGES = 320000
D_FEAT = 128
OUT_FEATS = 128

NC = 2
NS = 16
NW = NC * NS
EDGES_PER_W = N_EDGES // NW
BLK = 125
NBLK = EDGES_PER_W // BLK
ROWS_PER_TILE = N_NODES // NS
NBUF = 2
NIDX = 4
LANE = 16
CNT_STEPS = BLK // LANE
CNT_TAIL = BLK - CNT_STEPS * LANE


def _sc_segment_sum(feature, src_r, dst_r, zero_rows, zero_cnt):
    mesh = plsc.VectorSubcoreMesh(core_axis_name="c", subcore_axis_name="s")

    @functools.partial(
        pl.kernel,
        mesh=mesh,
        compiler_params=pltpu.CompilerParams(
            use_tc_tiling_on_sc=False, needs_layout_passes=False),
        out_type=(
            jax.ShapeDtypeStruct((NC, N_NODES, D_FEAT), jnp.float32),
            jax.ShapeDtypeStruct((NW, N_NODES), jnp.float32),
        ),
        scratch_types=[
            [pltpu.VMEM((2, BLK), jnp.int32)] * NIDX,
            [pltpu.VMEM((BLK, D_FEAT), jnp.float32)] * NBUF,
            pltpu.VMEM((N_NODES,), jnp.float32),
            pltpu.VMEM_SHARED((N_NODES, D_FEAT), jnp.float32),
            [pltpu.SemaphoreType.DMA] * NIDX,
            [pltpu.SemaphoreType.DMA] * NBUF,
            [pltpu.SemaphoreType.DMA] * NBUF,
        ],
    )
    def kern(feat_hbm, src_hbm, dst_hbm, zrows_hbm, zcnt_hbm,
             out_hbm, cnt_hbm, islot, rows, cnt_v, acc_sh,
             sem_i, sem_g, sem_s):
        c = lax.axis_index("c")
        s = lax.axis_index("s")
        wid = s * NC + c
        me_s = src_hbm.at[wid]
        me_d = dst_hbm.at[wid]

        def fetch_idx(j, q):
            pltpu.async_copy(me_s.at[j], islot[q].at[0], sem_i[q])
            pltpu.async_copy(me_d.at[j], islot[q].at[1], sem_i[q])

        def wait_idx(j, q):
            pltpu.make_async_copy(me_s.at[j], islot[q].at[0], sem_i[q]).wait()
            pltpu.make_async_copy(me_d.at[j], islot[q].at[1], sem_i[q]).wait()

        for q in range(NIDX - 1):
            fetch_idx(q, q)
        pltpu.sync_copy(zcnt_hbm, cnt_v)
        wait_idx(0, 0)
        pltpu.async_copy(feat_hbm.at[islot[0].at[0]], rows[0], sem_g[0])
        pltpu.sync_copy(zrows_hbm, acc_sh.at[pl.ds(s * ROWS_PER_TILE, ROWS_PER_TILE)])
        plsc.subcore_barrier()

        ones = jnp.ones((LANE,), jnp.float32)
        tail_mask = lax.iota(jnp.int32, LANE) >= (LANE - CNT_TAIL)

        def outer(k, carry):
            j0 = k * NIDX
            for q in range(NIDX):
                j = j0 + q
                b = q % NBUF
                pltpu.make_async_copy(feat_hbm.at[islot[q].at[0]], rows[b],
                                      sem_g[b]).wait()
                pltpu.async_copy(rows[b], acc_sh.at[islot[q].at[1]], sem_s[b],
                                 add=True)
                for t in range(CNT_STEPS):
                    idx = islot[q][1, pl.ds(t * LANE, LANE)]
                    plsc.addupdate_scatter(cnt_v, [idx], ones)
                idx = islot[q][1, pl.ds(BLK - LANE, LANE)]
                plsc.addupdate_scatter(cnt_v, [idx], ones, mask=tail_mask)

                bn = (q + 1) % NBUF
                qn = (q + 1) % NIDX

                @pl.when(j + 1 < NBLK)
                def _():
                    @pl.when(j >= 1)
                    def _():
                        pltpu.make_async_copy(
                            rows[bn], acc_sh.at[islot[qn].at[1]],
                            sem_s[bn]).wait()
                    wait_idx(j + 1, qn)
                    pltpu.async_copy(feat_hbm.at[islot[qn].at[0]], rows[bn],
                                     sem_g[bn])

                @pl.when(j + NIDX - 1 < NBLK)
                def _():
                    fetch_idx(j + NIDX - 1, (q + NIDX - 1) % NIDX)
            return carry

        lax.fori_loop(0, NBLK // NIDX, outer, 0)

        for b in range(NBUF):
            pltpu.make_async_copy(rows[b], acc_sh.at[islot[b].at[1]],
                                  sem_s[b]).wait()
        pltpu.sync_copy(cnt_v, cnt_hbm.at[wid])
        plsc.subcore_barrier()
        pltpu.sync_copy(
            acc_sh.at[pl.ds(s * ROWS_PER_TILE, ROWS_PER_TILE)],
            out_hbm.at[c].at[pl.ds(s * ROWS_PER_TILE, ROWS_PER_TILE)],
        )

    return kern(feature, src_r, dst_r, zero_rows, zero_cnt)


def _tc_linear_x(feature, w2t, b2d):
    blk_rows = 1000
    grid = (N_NODES // blk_rows,)

    def body(f_ref, w2_ref, b_ref, o_ref):
        o_ref[...] = (
            jnp.dot(f_ref[...], w2_ref[...], preferred_element_type=jnp.float32)
            + b_ref[...]
        )

    return pl.pallas_call(
        body,
        grid=grid,
        in_specs=[
            pl.BlockSpec((blk_rows, D_FEAT), lambda i: (i, 0)),
            pl.BlockSpec((D_FEAT, OUT_FEATS), lambda i: (0, 0)),
            pl.BlockSpec((1, OUT_FEATS), lambda i: (0, 0)),
        ],
        out_specs=pl.BlockSpec((blk_rows, OUT_FEATS), lambda i: (i, 0)),
        out_shape=jax.ShapeDtypeStruct((N_NODES, OUT_FEATS), jnp.float32),
    )(feature, w2t, b2d)


def _tc_mean_linear(partials, counts, xw2b, w1t):
    blk_rows = 1000
    grid = (N_NODES // blk_rows,)

    def body(p_ref, c_ref, x_ref, w1_ref, o_ref):
        summed = p_ref[0] + p_ref[1]
        cnt = c_ref[...]
        h = summed / jnp.maximum(cnt, 1.0)
        o_ref[...] = (
            jnp.dot(h, w1_ref[...], preferred_element_type=jnp.float32)
            + x_ref[...]
        )

    return pl.pallas_call(
        body,
        grid=grid,
        in_specs=[
            pl.BlockSpec((NC, blk_rows, D_FEAT), lambda i: (0, i, 0)),
            pl.BlockSpec((blk_rows, 1), lambda i: (i, 0)),
            pl.BlockSpec((blk_rows, OUT_FEATS), lambda i: (i, 0)),
            pl.BlockSpec((D_FEAT, OUT_FEATS), lambda i: (0, 0)),
        ],
        out_specs=pl.BlockSpec((blk_rows, OUT_FEATS), lambda i: (i, 0)),
        out_shape=jax.ShapeDtypeStruct((N_NODES, OUT_FEATS), jnp.float32),
    )(partials, counts, xw2b, w1t)


def kernel(feature, edge_index, W, b):
    ei = edge_index.astype(jnp.int32)
    src_r = ei[0].reshape(NW, NBLK, BLK)
    dst_r = ei[1].reshape(NW, NBLK, BLK)

    zero_rows = jnp.zeros((ROWS_PER_TILE, D_FEAT), jnp.float32)
    zero_cnt = jnp.zeros((N_NODES,), jnp.float32)

    w1t = W[:, :D_FEAT].T
    w2t = W[:, D_FEAT:].T
    b2d = b.reshape(1, OUT_FEATS)

    xw2b = _tc_linear_x(feature, w2t, b2d)
    partials, counts = _sc_segment_sum(feature, src_r, dst_r, zero_rows, zero_cnt)
    counts2d = counts.sum(axis=0).reshape(N_NODES, 1)
    return _tc_mean_linear(partials, counts2d, xw2b, w1t)


# speedup vs baseline: 1.7316x; 1.0074x over previous
import functools

import jax
import jax.numpy as jnp
from jax import lax
from jax.experimental import pallas as pl
from jax.experimental.pallas import tpu as pltpu
from jax.experimental.pallas import tpu_sc as plsc

N_NODES = 10000
N_EDGES = 320000
D_FEAT = 128
OUT_FEATS = 128

NC = 2
NS = 16
NW = NC * NS
EDGES_PER_W = N_EDGES // NW
BLK = 125
NBLK = EDGES_PER_W // BLK
ROWS_PER_TILE = N_NODES // NS
NBUF = 2
NIDX = 4
LANE = 16
CNT_STEPS = BLK // LANE
CNT_TAIL = BLK - CNT_STEPS * LANE


def _sc_segment_sum(feature, src_r, dst_r, zero_rows, zero_cnt):
    mesh = plsc.VectorSubcoreMesh(core_axis_name="c", subcore_axis_name="s")

    @functools.partial(
        pl.kernel,
        mesh=mesh,
        compiler_params=pltpu.CompilerParams(
            use_tc_tiling_on_sc=False, needs_layout_passes=False),
        out_type=(
            jax.ShapeDtypeStruct((NC, N_NODES, D_FEAT), jnp.float32),
            jax.ShapeDtypeStruct((NW, N_NODES), jnp.float32),
        ),
        scratch_types=[
            [pltpu.VMEM((2, BLK), jnp.int32)] * NIDX,
            [pltpu.VMEM((BLK, D_FEAT), jnp.float32)] * NBUF,
            pltpu.VMEM((N_NODES,), jnp.float32),
            pltpu.VMEM_SHARED((N_NODES, D_FEAT), jnp.float32),
            [pltpu.SemaphoreType.DMA] * NIDX,
            [pltpu.SemaphoreType.DMA] * NBUF,
            [pltpu.SemaphoreType.DMA] * NBUF,
        ],
    )
    def kern(feat_hbm, src_hbm, dst_hbm, zrows_hbm, zcnt_hbm,
             out_hbm, cnt_hbm, islot, rows, cnt_v, acc_sh,
             sem_i, sem_g, sem_s):
        c = lax.axis_index("c")
        s = lax.axis_index("s")
        wid = s * NC + c
        me_s = src_hbm.at[wid]
        me_d = dst_hbm.at[wid]

        def fetch_idx(j, q):
            pltpu.async_copy(me_s.at[j], islot[q].at[0], sem_i[q])
            pltpu.async_copy(me_d.at[j], islot[q].at[1], sem_i[q])

        def wait_idx(j, q):
            pltpu.make_async_copy(me_s.at[j], islot[q].at[0], sem_i[q]).wait()
            pltpu.make_async_copy(me_d.at[j], islot[q].at[1], sem_i[q]).wait()

        for q in range(NIDX - 1):
            fetch_idx(q, q)
        pltpu.sync_copy(zcnt_hbm, cnt_v)
        wait_idx(0, 0)
        pltpu.async_copy(feat_hbm.at[islot[0].at[0]], rows[0], sem_g[0])
        pltpu.sync_copy(zrows_hbm, acc_sh.at[pl.ds(s * ROWS_PER_TILE, ROWS_PER_TILE)])
        plsc.subcore_barrier()

        ones = jnp.ones((LANE,), jnp.float32)
        tail_mask = lax.iota(jnp.int32, LANE) >= (LANE - CNT_TAIL)

        def outer(k, carry):
            j0 = k * NIDX
            for q in range(NIDX):
                j = j0 + q
                b = q % NBUF
                pltpu.make_async_copy(feat_hbm.at[islot[q].at[0]], rows[b],
                                      sem_g[b]).wait()
                pltpu.async_copy(rows[b], acc_sh.at[islot[q].at[1]], sem_s[b],
                                 add=True)
                for t in range(CNT_STEPS):
                    idx = islot[q][1, pl.ds(t * LANE, LANE)]
                    plsc.addupdate_scatter(cnt_v, [idx], ones)
                idx = islot[q][1, pl.ds(BLK - LANE, LANE)]
                plsc.addupdate_scatter(cnt_v, [idx], ones, mask=tail_mask)

                bn = (q + 1) % NBUF
                qn = (q + 1) % NIDX

                @pl.when(j + 1 < NBLK)
                def _():
                    @pl.when(j >= 1)
                    def _():
                        pltpu.make_async_copy(
                            rows[bn], acc_sh.at[islot[qn].at[1]],
                            sem_s[bn]).wait()
                    wait_idx(j + 1, qn)
                    pltpu.async_copy(feat_hbm.at[islot[qn].at[0]], rows[bn],
                                     sem_g[bn])

                @pl.when(j + NIDX - 1 < NBLK)
                def _():
                    fetch_idx(j + NIDX - 1, (q + NIDX - 1) % NIDX)
            return carry

        lax.fori_loop(0, NBLK // NIDX, outer, 0)

        for b in range(NBUF):
            pltpu.make_async_copy(rows[b], acc_sh.at[islot[b].at[1]],
                                  sem_s[b]).wait()
        pltpu.sync_copy(cnt_v, cnt_hbm.at[wid])
        plsc.subcore_barrier()
        pltpu.sync_copy(
            acc_sh.at[pl.ds(s * ROWS_PER_TILE, ROWS_PER_TILE)],
            out_hbm.at[c].at[pl.ds(s * ROWS_PER_TILE, ROWS_PER_TILE)],
        )

    return kern(feature, src_r, dst_r, zero_rows, zero_cnt)


def _tc_mean_linear(partials, counts2d, feature, w1t, w2t, b2d):
    blk_rows = 1000
    grid = (N_NODES // blk_rows,)

    def body(p_ref, c_ref, f_ref, w1_ref, w2_ref, b_ref, o_ref):
        summed = p_ref[0] + p_ref[1]
        cnt = c_ref[...]
        h = summed / jnp.maximum(cnt, 1.0)
        o_ref[...] = (
            jnp.dot(h, w1_ref[...], preferred_element_type=jnp.float32)
            + jnp.dot(f_ref[...], w2_ref[...], preferred_element_type=jnp.float32)
            + b_ref[...]
        )

    return pl.pallas_call(
        body,
        grid=grid,
        in_specs=[
            pl.BlockSpec((NC, blk_rows, D_FEAT), lambda i: (0, i, 0)),
            pl.BlockSpec((blk_rows, 1), lambda i: (i, 0)),
            pl.BlockSpec((blk_rows, D_FEAT), lambda i: (i, 0)),
            pl.BlockSpec((D_FEAT, OUT_FEATS), lambda i: (0, 0)),
            pl.BlockSpec((D_FEAT, OUT_FEATS), lambda i: (0, 0)),
            pl.BlockSpec((1, OUT_FEATS), lambda i: (0, 0)),
        ],
        out_specs=pl.BlockSpec((blk_rows, OUT_FEATS), lambda i: (i, 0)),
        out_shape=jax.ShapeDtypeStruct((N_NODES, OUT_FEATS), jnp.float32),
    )(partials, counts2d, feature, w1t, w2t, b2d)


def kernel(feature, edge_index, W, b):
    ei = edge_index.astype(jnp.int32)
    src_r = ei[0].reshape(NW, NBLK, BLK)
    dst_r = ei[1].reshape(NW, NBLK, BLK)

    zero_rows = jnp.zeros((ROWS_PER_TILE, D_FEAT), jnp.float32)
    zero_cnt = jnp.zeros((N_NODES,), jnp.float32)

    w1t = W[:, :D_FEAT].T
    w2t = W[:, D_FEAT:].T
    b2d = b.reshape(1, OUT_FEATS)

    partials, counts = _sc_segment_sum(feature, src_r, dst_r, zero_rows, zero_cnt)
    counts2d = counts.sum(axis=0).reshape(N_NODES, 1)
    return _tc_mean_linear(partials, counts2d, feature, w1t, w2t, b2d)


# speedup vs baseline: 1.7322x; 1.0003x over previous
import functools

import jax
import jax.numpy as jnp
from jax import lax
from jax.experimental import pallas as pl
from jax.experimental.pallas import tpu as pltpu
from jax.experimental.pallas import tpu_sc as plsc

N_NODES = 10000
N_EDGES = 320000
D_FEAT = 128
OUT_FEATS = 128

NC = 2
NS = 16
NW = NC * NS
EDGES_PER_W = N_EDGES // NW
BLK = 125
NBLK = EDGES_PER_W // BLK
ROWS_PER_TILE = N_NODES // NS
NBUF = 2
BANK = 8
NBANK = NBLK // BANK
LANE = 16
CNT_STEPS = BLK // LANE
CNT_TAIL = BLK - CNT_STEPS * LANE


def _sc_segment_sum(feature, src_r, dst_r, zero_rows, zero_cnt):
    mesh = plsc.VectorSubcoreMesh(core_axis_name="c", subcore_axis_name="s")

    @functools.partial(
        pl.kernel,
        mesh=mesh,
        compiler_params=pltpu.CompilerParams(
            use_tc_tiling_on_sc=False, needs_layout_passes=False),
        out_type=(
            jax.ShapeDtypeStruct((NC, N_NODES, D_FEAT), jnp.float32),
            jax.ShapeDtypeStruct((NW, N_NODES), jnp.float32),
        ),
        scratch_types=[
            [pltpu.VMEM((BANK, BLK), jnp.int32)] * 2,
            [pltpu.VMEM((BANK, BLK), jnp.int32)] * 2,
            [pltpu.VMEM((BLK, D_FEAT), jnp.float32)] * NBUF,
            pltpu.VMEM((N_NODES,), jnp.float32),
            pltpu.VMEM_SHARED((N_NODES, D_FEAT), jnp.float32),
            [pltpu.SemaphoreType.DMA] * 2,
            [pltpu.SemaphoreType.DMA] * NBUF,
            [pltpu.SemaphoreType.DMA] * NBUF,
        ],
    )
    def kern(feat_hbm, src_hbm, dst_hbm, zrows_hbm, zcnt_hbm,
             out_hbm, cnt_hbm, sbank, dbank, rows, cnt_v, acc_sh,
             sem_ib, sem_g, sem_s):
        c = lax.axis_index("c")
        s = lax.axis_index("s")
        wid = s * NC + c
        me_s = src_hbm.at[wid]
        me_d = dst_hbm.at[wid]

        def fetch_bank(n, slot):
            pltpu.async_copy(me_s.at[n], sbank[slot], sem_ib[slot])
            pltpu.async_copy(me_d.at[n], dbank[slot], sem_ib[slot])

        def wait_bank(n, slot):
            pltpu.make_async_copy(me_s.at[n], sbank[slot], sem_ib[slot]).wait()
            pltpu.make_async_copy(me_d.at[n], dbank[slot], sem_ib[slot]).wait()

        fetch_bank(0, 0)
        fetch_bank(1, 1)
        pltpu.sync_copy(zcnt_hbm, cnt_v)
        wait_bank(0, 0)
        pltpu.async_copy(feat_hbm.at[sbank[0].at[0]], rows[0], sem_g[0])
        pltpu.sync_copy(zrows_hbm, acc_sh.at[pl.ds(s * ROWS_PER_TILE, ROWS_PER_TILE)])
        plsc.subcore_barrier()

        ones = jnp.ones((LANE,), jnp.float32)
        tail_mask = lax.iota(jnp.int32, LANE) >= (LANE - CNT_TAIL)

        def outer(m, carry):
            for half in range(2):
                k = 2 * m + half
                sb, db = sbank[half], dbank[half]
                sb_n = sbank[half ^ 1]
                for q in range(BANK):
                    j = k * BANK + q
                    b = q % NBUF
                    bn = (q + 1) % NBUF
                    pltpu.make_async_copy(feat_hbm.at[sb.at[q]], rows[b],
                                          sem_g[b]).wait()
                    pltpu.async_copy(rows[b], acc_sh.at[db.at[q]], sem_s[b],
                                     add=True)
                    for t in range(CNT_STEPS):
                        idx = db[q, pl.ds(t * LANE, LANE)]
                        plsc.addupdate_scatter(cnt_v, [idx], ones)
                    idx = db[q, pl.ds(BLK - LANE, LANE)]
                    plsc.addupdate_scatter(cnt_v, [idx], ones, mask=tail_mask)

                    if q < BANK - 1:
                        @pl.when(j >= 1)
                        def _():
                            pltpu.make_async_copy(
                                rows[bn], acc_sh.at[db.at[q + 1]],
                                sem_s[bn]).wait()
                        pltpu.async_copy(feat_hbm.at[sb.at[q + 1]], rows[bn],
                                         sem_g[bn])
                    else:
                        @pl.when(j + 1 < NBLK)
                        def _():
                            pltpu.make_async_copy(
                                rows[bn], acc_sh.at[db.at[q]],
                                sem_s[bn]).wait()
                            wait_bank(k + 1, half ^ 1)
                            pltpu.async_copy(feat_hbm.at[sb_n.at[0]],
                                             rows[bn], sem_g[bn])

                    if q == 2:
                        @pl.when((k >= 1) & (k + 1 < NBANK))
                        def _():
                            fetch_bank(k + 1, half ^ 1)
            return carry

        lax.fori_loop(0, NBANK // 2, outer, 0)

        for b in range(NBUF):
            pltpu.make_async_copy(rows[b], acc_sh.at[dbank[1].at[b]],
                                  sem_s[b]).wait()
        pltpu.sync_copy(cnt_v, cnt_hbm.at[wid])
        plsc.subcore_barrier()
        pltpu.sync_copy(
            acc_sh.at[pl.ds(s * ROWS_PER_TILE, ROWS_PER_TILE)],
            out_hbm.at[c].at[pl.ds(s * ROWS_PER_TILE, ROWS_PER_TILE)],
        )

    return kern(feature, src_r, dst_r, zero_rows, zero_cnt)


def _tc_mean_linear(partials, counts2d, feature, w1t, w2t, b2d):
    blk_rows = 1000
    grid = (N_NODES // blk_rows,)

    def body(p_ref, c_ref, f_ref, w1_ref, w2_ref, b_ref, o_ref):
        summed = p_ref[0] + p_ref[1]
        cnt = c_ref[...]
        h = summed / jnp.maximum(cnt, 1.0)
        o_ref[...] = (
            jnp.dot(h, w1_ref[...], preferred_element_type=jnp.float32)
            + jnp.dot(f_ref[...], w2_ref[...], preferred_element_type=jnp.float32)
            + b_ref[...]
        )

    return pl.pallas_call(
        body,
        grid=grid,
        in_specs=[
            pl.BlockSpec((NC, blk_rows, D_FEAT), lambda i: (0, i, 0)),
            pl.BlockSpec((blk_rows, 1), lambda i: (i, 0)),
            pl.BlockSpec((blk_rows, D_FEAT), lambda i: (i, 0)),
            pl.BlockSpec((D_FEAT, OUT_FEATS), lambda i: (0, 0)),
            pl.BlockSpec((D_FEAT, OUT_FEATS), lambda i: (0, 0)),
            pl.BlockSpec((1, OUT_FEATS), lambda i: (0, 0)),
        ],
        out_specs=pl.BlockSpec((blk_rows, OUT_FEATS), lambda i: (i, 0)),
        out_shape=jax.ShapeDtypeStruct((N_NODES, OUT_FEATS), jnp.float32),
    )(partials, counts2d, feature, w1t, w2t, b2d)


def kernel(feature, edge_index, W, b):
    ei = edge_index.astype(jnp.int32)
    src_r = ei[0].reshape(NW, NBANK, BANK, BLK)
    dst_r = ei[1].reshape(NW, NBANK, BANK, BLK)

    zero_rows = jnp.zeros((ROWS_PER_TILE, D_FEAT), jnp.float32)
    zero_cnt = jnp.zeros((N_NODES,), jnp.float32)

    w1t = W[:, :D_FEAT].T
    w2t = W[:, D_FEAT:].T
    b2d = b.reshape(1, OUT_FEATS)

    partials, counts = _sc_segment_sum(feature, src_r, dst_r, zero_rows, zero_cnt)
    counts2d = counts.sum(axis=0).reshape(N_NODES, 1)
    return _tc_mean_linear(partials, counts2d, feature, w1t, w2t, b2d)


# speedup vs baseline: 1.7707x; 1.0223x over previous
import functools

import jax
import jax.numpy as jnp
from jax import lax
from jax.experimental import pallas as pl
from jax.experimental.pallas import tpu as pltpu
from jax.experimental.pallas import tpu_sc as plsc

N_NODES = 10000
N_EDGES = 320000
D_FEAT = 128
OUT_FEATS = 128

NC = 2
NS = 16
NW = NC * NS
EDGES_PER_W = N_EDGES // NW
BLK = 125
NBLK = EDGES_PER_W // BLK
ROWS_PER_TILE = N_NODES // NS
NBUF = 2
BANK = 8
NBANK = NBLK // BANK
LANE = 16
CNT_STEPS = BLK // LANE
CNT_TAIL = BLK - CNT_STEPS * LANE
CNT_BLK = 1000
NCB = N_NODES // CNT_BLK


def _sc_segment_sum(feature, src_r, dst_r, zero_rows, zero_cnt):
    mesh = plsc.VectorSubcoreMesh(core_axis_name="c", subcore_axis_name="s")

    @functools.partial(
        pl.kernel,
        mesh=mesh,
        compiler_params=pltpu.CompilerParams(
            use_tc_tiling_on_sc=False, needs_layout_passes=False),
        out_type=(
            jax.ShapeDtypeStruct((NC, N_NODES, D_FEAT), jnp.float32),
            jax.ShapeDtypeStruct((NCB, NW, CNT_BLK), jnp.float32),
        ),
        scratch_types=[
            [pltpu.VMEM((BANK, BLK), jnp.int32)] * 2,
            [pltpu.VMEM((BANK, BLK), jnp.int32)] * 2,
            [pltpu.VMEM((BLK, D_FEAT), jnp.float32)] * NBUF,
            pltpu.VMEM((N_NODES,), jnp.float32),
            pltpu.VMEM_SHARED((N_NODES, D_FEAT), jnp.float32),
            [pltpu.SemaphoreType.DMA] * 2,
            [pltpu.SemaphoreType.DMA] * NBUF,
            [pltpu.SemaphoreType.DMA] * NBUF,
        ],
    )
    def kern(feat_hbm, src_hbm, dst_hbm, zrows_hbm, zcnt_hbm,
             out_hbm, cnt_hbm, sbank, dbank, rows, cnt_v, acc_sh,
             sem_ib, sem_g, sem_s):
        c = lax.axis_index("c")
        s = lax.axis_index("s")
        wid = s * NC + c
        me_s = src_hbm.at[wid]
        me_d = dst_hbm.at[wid]

        def fetch_bank(n, slot):
            pltpu.async_copy(me_s.at[n], sbank[slot], sem_ib[slot])
            pltpu.async_copy(me_d.at[n], dbank[slot], sem_ib[slot])

        def wait_bank(n, slot):
            pltpu.make_async_copy(me_s.at[n], sbank[slot], sem_ib[slot]).wait()
            pltpu.make_async_copy(me_d.at[n], dbank[slot], sem_ib[slot]).wait()

        fetch_bank(0, 0)
        fetch_bank(1, 1)
        pltpu.sync_copy(zcnt_hbm, cnt_v)
        wait_bank(0, 0)
        pltpu.async_copy(feat_hbm.at[sbank[0].at[0]], rows[0], sem_g[0])
        pltpu.sync_copy(zrows_hbm, acc_sh.at[pl.ds(s * ROWS_PER_TILE, ROWS_PER_TILE)])
        plsc.subcore_barrier()

        ones = jnp.ones((LANE,), jnp.float32)
        tail_mask = lax.iota(jnp.int32, LANE) >= (LANE - CNT_TAIL)

        def outer(m, carry):
            for half in range(2):
                k = 2 * m + half
                sb, db = sbank[half], dbank[half]
                sb_n = sbank[half ^ 1]
                for q in range(BANK):
                    j = k * BANK + q
                    b = q % NBUF
                    bn = (q + 1) % NBUF
                    pltpu.make_async_copy(feat_hbm.at[sb.at[q]], rows[b],
                                          sem_g[b]).wait()
                    pltpu.async_copy(rows[b], acc_sh.at[db.at[q]], sem_s[b],
                                     add=True)
                    for t in range(CNT_STEPS):
                        idx = db[q, pl.ds(t * LANE, LANE)]
                        plsc.addupdate_scatter(cnt_v, [idx], ones)
                    idx = db[q, pl.ds(BLK - LANE, LANE)]
                    plsc.addupdate_scatter(cnt_v, [idx], ones, mask=tail_mask)

                    if q < BANK - 1:
                        @pl.when(j >= 1)
                        def _():
                            pltpu.make_async_copy(
                                rows[bn], acc_sh.at[db.at[q + 1]],
                                sem_s[bn]).wait()
                        pltpu.async_copy(feat_hbm.at[sb.at[q + 1]], rows[bn],
                                         sem_g[bn])
                    else:
                        @pl.when(j + 1 < NBLK)
                        def _():
                            pltpu.make_async_copy(
                                rows[bn], acc_sh.at[db.at[q]],
                                sem_s[bn]).wait()
                            wait_bank(k + 1, half ^ 1)
                            pltpu.async_copy(feat_hbm.at[sb_n.at[0]],
                                             rows[bn], sem_g[bn])

                    if q == 2:
                        @pl.when((k >= 1) & (k + 1 < NBANK))
                        def _():
                            fetch_bank(k + 1, half ^ 1)
            return carry

        lax.fori_loop(0, NBANK // 2, outer, 0)

        for b in range(NBUF):
            pltpu.make_async_copy(rows[b], acc_sh.at[dbank[1].at[b]],
                                  sem_s[b]).wait()
        for t in range(NCB):
            pltpu.async_copy(cnt_v.at[pl.ds(t * CNT_BLK, CNT_BLK)],
                             cnt_hbm.at[t].at[wid], sem_ib[0])
        for t in range(NCB):
            pltpu.make_async_copy(cnt_v.at[pl.ds(t * CNT_BLK, CNT_BLK)],
                                  cnt_hbm.at[t].at[wid], sem_ib[0]).wait()
        plsc.subcore_barrier()
        pltpu.sync_copy(
            acc_sh.at[pl.ds(s * ROWS_PER_TILE, ROWS_PER_TILE)],
            out_hbm.at[c].at[pl.ds(s * ROWS_PER_TILE, ROWS_PER_TILE)],
        )

    return kern(feature, src_r, dst_r, zero_rows, zero_cnt)


def _tc_mean_linear(partials, counts, feature, w1t, w2t, b2d):
    blk_rows = 1000
    grid = (N_NODES // blk_rows,)

    def body(p_ref, c_ref, f_ref, w1_ref, w2_ref, b_ref, o_ref):
        summed = p_ref[0] + p_ref[1]
        cnt = jnp.sum(c_ref[0], axis=0).reshape(blk_rows, 1)
        h = summed / jnp.maximum(cnt, 1.0)
        o_ref[...] = (
            jnp.dot(h, w1_ref[...], preferred_element_type=jnp.float32)
            + jnp.dot(f_ref[...], w2_ref[...], preferred_element_type=jnp.float32)
            + b_ref[...]
        )

    return pl.pallas_call(
        body,
        grid=grid,
        in_specs=[
            pl.BlockSpec((NC, blk_rows, D_FEAT), lambda i: (0, i, 0)),
            pl.BlockSpec((1, NW, CNT_BLK), lambda i: (i, 0, 0)),
            pl.BlockSpec((blk_rows, D_FEAT), lambda i: (i, 0)),
            pl.BlockSpec((D_FEAT, OUT_FEATS), lambda i: (0, 0)),
            pl.BlockSpec((D_FEAT, OUT_FEATS), lambda i: (0, 0)),
            pl.BlockSpec((1, OUT_FEATS), lambda i: (0, 0)),
        ],
        out_specs=pl.BlockSpec((blk_rows, OUT_FEATS), lambda i: (i, 0)),
        out_shape=jax.ShapeDtypeStruct((N_NODES, OUT_FEATS), jnp.float32),
    )(partials, counts, feature, w1t, w2t, b2d)


def kernel(feature, edge_index, W, b):
    ei = edge_index.astype(jnp.int32)
    src_r = ei[0].reshape(NW, NBANK, BANK, BLK)
    dst_r = ei[1].reshape(NW, NBANK, BANK, BLK)

    zero_rows = jnp.zeros((ROWS_PER_TILE, D_FEAT), jnp.float32)
    zero_cnt = jnp.zeros((N_NODES,), jnp.float32)

    w1t = W[:, :D_FEAT].T
    w2t = W[:, D_FEAT:].T
    b2d = b.reshape(1, OUT_FEATS)

    partials, counts = _sc_segment_sum(feature, src_r, dst_r, zero_rows, zero_cnt)
    return _tc_mean_linear(partials, counts, feature, w1t, w2t, b2d)
